# baseline (speedup 1.0000x reference)
import functools

import numpy as np
import jax
import jax.numpy as jnp
from jax import lax
from jax.experimental import pallas as pl
from jax.experimental.pallas import tpu as pltpu
from jax.experimental.pallas import tpu_sc as plsc

N_NODES = 10000
N_EDGES = 160000
D = 256
H = 128
RANK = 32
B = 4096

NC = 2
NS = 16
L = 16

EPT = N_EDGES // NS
C = 64
NBUF = 3
NCH = 159
EPTP = NCH * C
ZR = 400
NZCH = N_NODES // ZR
WR = 80
NWCH = N_NODES // WR

_mesh = plsc.VectorSubcoreMesh(core_axis_name="c", subcore_axis_name="s")


@functools.partial(
    pl.kernel,
    out_type=jax.ShapeDtypeStruct((NC, N_NODES, H), jnp.float32),
    mesh=_mesh,
    scratch_types=[
        pltpu.VMEM((NBUF, 2, C), jnp.int32),
        pltpu.VMEM((NBUF, C), jnp.float32),
        pltpu.VMEM((NBUF, C, H), jnp.bfloat16),
        pltpu.VMEM((NBUF, C, H), jnp.float32),
        pltpu.VMEM_SHARED((N_NODES, H), jnp.float32),
        pltpu.SemaphoreType.DMA((NBUF,)),
        pltpu.SemaphoreType.DMA((NBUF,)),
        pltpu.SemaphoreType.DMA((NBUF,)),
    ],
    compiler_params=pltpu.CompilerParams(use_tc_tiling_on_sc=False),
)
def _edge_agg(x2, edges, ws, zrows, agg, ec_v, w_v, brows_v, frows_v, acc,
              gsem, gsem2, ssem):
    c = lax.axis_index("c")
    s = lax.axis_index("s")

    def zcp(k, carry):
        q = s + k * NS
        pltpu.sync_copy(zrows, acc.at[pl.ds(q * ZR, ZR)])
        return carry

    lax.fori_loop(0, (NZCH - s + NS - 1) // NS, zcp, 0)
    plsc.subcore_barrier()

    CH = C // 2

    def stage(j, b):
        pltpu.sync_copy(edges.at[c, s, j], ec_v.at[b])
        pltpu.async_copy(ws.at[s, j], w_v.at[b], gsem.at[b])
        pltpu.async_copy(x2.at[ec_v.at[b, 0, pl.ds(0, CH)]],
                         brows_v.at[b, pl.ds(0, CH)], gsem.at[b])
        pltpu.async_copy(x2.at[ec_v.at[b, 0, pl.ds(CH, CH)]],
                         brows_v.at[b, pl.ds(CH, CH)], gsem2.at[b])

    def wait_gather(b):
        pltpu.make_async_copy(ws.at[0, 0], w_v.at[b], gsem.at[b]).wait()
        pltpu.make_async_copy(x2.at[pl.ds(0, CH)],
                              brows_v.at[b, pl.ds(0, CH)], gsem.at[b]).wait()
        pltpu.make_async_copy(x2.at[pl.ds(0, CH)],
                              brows_v.at[b, pl.ds(CH, CH)], gsem2.at[b]).wait()

    def wait_scatter(b):
        pltpu.make_async_copy(frows_v.at[b], acc.at[pl.ds(0, C)],
                              ssem.at[b]).wait()

    stage(0, 0)
    stage(1, 1)

    def process(j, b):
        wait_gather(b)

        w32 = None
        for e in range(C):
            l = e % (2 * L)
            if l == 0:
                w32 = w_v[b, pl.ds(e, 2 * L)]
            wsp2 = w32.at[jnp.full((2 * L,), l, jnp.int32)].get(
                mode="promise_in_bounds")
            for k in range(H // (2 * L)):
                sl = pl.ds(k * 2 * L, 2 * L)
                v = brows_v[b, e, sl].astype(jnp.float32)
                frows_v[b, e, sl] = v * wsp2
        pltpu.async_copy(frows_v.at[b], acc.at[ec_v.at[b, 1]], ssem.at[b],
                         add=True)

    def chunk3(i, carry):
        for t in range(NBUF):
            j = NBUF * i + t
            b2 = (t + 2) % NBUF

            @pl.when(j + 2 < NCH)
            def _(j=j, b2=b2):
                @pl.when(j >= 1)
                def _():
                    wait_scatter(b2)
                stage(j + 2, b2)

            process(j, t)
        return carry

    lax.fori_loop(0, NCH // NBUF, chunk3, 0)
    for b in range(NBUF):
        wait_scatter(b)
    plsc.subcore_barrier()

    def wcp(k, carry):
        q = s + k * NS
        pltpu.sync_copy(acc.at[pl.ds(q * WR, WR)], agg.at[c, pl.ds(q * WR, WR)])
        return carry

    lax.fori_loop(0, (NWCH - s + NS - 1) // NS, wcp, 0)


BPW = B // (NC * NS)


@functools.partial(
    pl.kernel,
    out_type=jax.ShapeDtypeStruct((B, D), jnp.float32),
    mesh=_mesh,
    scratch_types=[
        pltpu.VMEM((BPW,), jnp.int32),
        pltpu.VMEM((BPW, D), jnp.float32),
        pltpu.SemaphoreType.DMA,
    ],
)
def _batch_gather(table, idx, out, idx_v, rows_v, sem):
    wid = lax.axis_index("s") * NC + lax.axis_index("c")
    base = wid * BPW
    pltpu.sync_copy(idx.at[pl.ds(base, BPW)], idx_v)
    pltpu.async_copy(table.at[idx_v], rows_v, sem).wait()
    pltpu.sync_copy(rows_v, out.at[pl.ds(base, BPW)])


TN = 1000


def _mm1_body(a_ref, w_ref, b_ref, o_ref):
    a = a_ref[...]
    x = jnp.concatenate([a[0], a[1]], axis=1)
    y = jnp.dot(x, w_ref[...], preferred_element_type=jnp.float32) + b_ref[...]
    y = jnp.maximum(y, 0.0).astype(jnp.bfloat16)
    o_ref[0] = y[:, :H]
    o_ref[1] = y[:, H:]


_mm1 = pl.pallas_call(
    _mm1_body,
    grid=(N_NODES // TN,),
    in_specs=[
        pl.BlockSpec((NC, TN, H), lambda i: (0, i, 0)),
        pl.BlockSpec((D, D), lambda i: (0, 0)),
        pl.BlockSpec((1, D), lambda i: (0, 0)),
    ],
    out_specs=pl.BlockSpec((NC, TN, H), lambda i: (0, i, 0)),
    out_shape=jax.ShapeDtypeStruct((NC, N_NODES, H), jnp.bfloat16),
)


def _mm2_body(a_ref, pa_ref, w_ref, pb_ref, b_ref, o_ref):
    a = a_ref[...]
    x = jnp.concatenate([a[0], a[1]], axis=1)
    y = jnp.dot(x, w_ref[...], preferred_element_type=jnp.float32)
    y = y + jnp.dot(pa_ref[...], pb_ref[...], preferred_element_type=jnp.float32)
    o_ref[...] = y + b_ref[...]


_mm2 = pl.pallas_call(
    _mm2_body,
    grid=(N_NODES // TN,),
    in_specs=[
        pl.BlockSpec((NC, TN, H), lambda i: (0, i, 0)),
        pl.BlockSpec((TN, RANK), lambda i: (i, 0)),
        pl.BlockSpec((D, D), lambda i: (0, 0)),
        pl.BlockSpec((RANK, D), lambda i: (0, 0)),
        pl.BlockSpec((1, D), lambda i: (0, 0)),
    ],
    out_specs=pl.BlockSpec((TN, D), lambda i: (i, 0)),
    out_shape=jax.ShapeDtypeStruct((N_NODES, D), jnp.float32),
)


SB = 1024


def _sel_body(iv_ref, g_ref, base_ref, oov_ref, o_ref):
    m = iv_ref[...] > 0
    o_ref[...] = jnp.where(m, g_ref[...], base_ref[...] + oov_ref[...])


_sel = pl.pallas_call(
    _sel_body,
    grid=(B // SB,),
    in_specs=[
        pl.BlockSpec((SB, 1), lambda i: (i, 0)),
        pl.BlockSpec((SB, D), lambda i: (i, 0)),
        pl.BlockSpec((SB, D), lambda i: (i, 0)),
        pl.BlockSpec((1, D), lambda i: (0, 0)),
    ],
    out_specs=pl.BlockSpec((SB, D), lambda i: (i, 0)),
    out_shape=jax.ShapeDtypeStruct((B, D), jnp.float32),
)


_PERM128 = np.arange(128).reshape(4, 2, 16).transpose(0, 2, 1).reshape(128)
_P256 = np.concatenate([_PERM128, 128 + _PERM128])


def kernel(base_embedding, node_idx, in_vocab, edge_index, edge_weight, emb,
           W1, b1, W2, b2, pert_A, pert_B, oov_weight):
    src = edge_index[0].astype(jnp.int32)
    dst = edge_index[1].astype(jnp.int32)
    pad = EPTP - EPT
    packed = jnp.stack([src, dst], axis=0).reshape(2, NS, EPT)
    packed = jnp.pad(packed, ((0, 0), (0, 0), (0, pad)))
    packed = packed.reshape(2, NS, NCH, C)
    packed = jnp.transpose(packed, (1, 2, 0, 3))
    off = jnp.array([N_NODES, 0], jnp.int32)[None, None, :, None]
    edges = jnp.stack([packed, packed + off], axis=0)
    ws = jnp.pad(edge_weight.reshape(NS, EPT),
                 ((0, 0), (0, pad))).reshape(NS, NCH, C)

    emb2 = jnp.concatenate([emb[:, _P256[:H]], emb[:, _P256[H:]]],
                           axis=0).astype(jnp.bfloat16)
    W1p = W1[_P256, :][:, _P256]
    b1p = b1[_P256]
    W2p = W2[_P256, :]

    zrows = jnp.zeros((ZR, H), jnp.float32)

    agg1 = _edge_agg(emb2, edges, ws, zrows)
    h1 = _mm1(agg1, W1p, b1p.reshape(1, D))
    agg2 = _edge_agg(h1.reshape(NC * N_NODES, H), edges, ws, zrows)
    h2 = _mm2(agg2, pert_A, W2p, pert_B, b2.reshape(1, D))
    g = _batch_gather(h2, node_idx.astype(jnp.int32))
    out = _sel(in_vocab.astype(jnp.int32).reshape(B, 1), g,
               base_embedding, oov_weight)
    return out


# speedup vs baseline: 1.2088x; 1.2088x over previous
import functools

import numpy as np
import jax
import jax.numpy as jnp
from jax import lax
from jax.experimental import pallas as pl
from jax.experimental.pallas import tpu as pltpu
from jax.experimental.pallas import tpu_sc as plsc

N_NODES = 10000
N_EDGES = 160000
D = 256
H = 128
RANK = 32
B = 4096

NC = 2
NS = 16
L = 16

EPT = N_EDGES // NS
C = 64
NBUF = 3
NCH = 159
EPTP = NCH * C
ZR = 400
NZCH = N_NODES // ZR
WR = 80
NWCH = N_NODES // WR

_mesh = plsc.VectorSubcoreMesh(core_axis_name="c", subcore_axis_name="s")


@functools.partial(
    pl.kernel,
    out_type=jax.ShapeDtypeStruct((NC, N_NODES, H), jnp.float32),
    mesh=_mesh,
    scratch_types=[
        pltpu.VMEM((NBUF, 2, C), jnp.int32),
        pltpu.VMEM((NBUF, C), jnp.float32),
        pltpu.VMEM((NBUF, C, H), jnp.bfloat16),
        pltpu.VMEM((NBUF, C, H), jnp.float32),
        pltpu.VMEM_SHARED((N_NODES, H), jnp.float32),
        pltpu.SemaphoreType.DMA((NBUF,)),
        pltpu.SemaphoreType.DMA((NBUF,)),
        pltpu.SemaphoreType.DMA((NBUF,)),
    ],
    compiler_params=pltpu.CompilerParams(use_tc_tiling_on_sc=False),
)
def _edge_agg(x2, edges, ws, zrows, agg, ec_v, w_v, brows_v, frows_v, acc,
              esem, gsem, ssem):
    c = lax.axis_index("c")
    s = lax.axis_index("s")

    def zcp(k, carry):
        q = s + k * NS
        pltpu.sync_copy(zrows, acc.at[pl.ds(q * ZR, ZR)])
        return carry

    lax.fori_loop(0, (NZCH - s + NS - 1) // NS, zcp, 0)
    plsc.subcore_barrier()

    def start_ec(j, b):
        pltpu.async_copy(edges.at[c, s, j], ec_v.at[b], esem.at[b])
        pltpu.async_copy(ws.at[s, j], w_v.at[b], esem.at[b])

    def wait_ec(b):
        pltpu.make_async_copy(edges.at[0, 0, 0], ec_v.at[b],
                              esem.at[b]).wait()
        pltpu.make_async_copy(ws.at[0, 0], w_v.at[b], esem.at[b]).wait()

    def start_gather(b):
        pltpu.async_copy(x2.at[ec_v.at[b, 0]], brows_v.at[b], gsem.at[b])

    def wait_gather(b):
        pltpu.make_async_copy(x2.at[pl.ds(0, C)], brows_v.at[b],
                              gsem.at[b]).wait()

    def wait_scatter(b):
        pltpu.make_async_copy(frows_v.at[b], acc.at[pl.ds(0, C)],
                              ssem.at[b]).wait()

    start_ec(0, 0)
    start_ec(1, 1)
    wait_ec(0)
    start_gather(0)

    def process(j, b):
        wait_gather(b)

        w32 = None
        for e in range(C):
            l = e % (2 * L)
            if l == 0:
                w32 = w_v[b, pl.ds(e, 2 * L)]
            wsp2 = w32.at[jnp.full((2 * L,), l, jnp.int32)].get(
                mode="promise_in_bounds")
            for k in range(H // (2 * L)):
                sl = pl.ds(k * 2 * L, 2 * L)
                v = brows_v[b, e, sl].astype(jnp.float32)
                frows_v[b, e, sl] = v * wsp2
        pltpu.async_copy(frows_v.at[b], acc.at[ec_v.at[b, 1]], ssem.at[b],
                         add=True)

    def chunk3(i, carry):
        for t in range(NBUF):
            j = NBUF * i + t
            b1 = (t + 1) % NBUF
            b2 = (t + 2) % NBUF

            @pl.when(j + 2 < NCH)
            def _(j=j, b2=b2):
                @pl.when(j >= 1)
                def _():
                    wait_scatter(b2)
                start_ec(j + 2, b2)

            @pl.when(j + 1 < NCH)
            def _(b1=b1):
                wait_ec(b1)
                start_gather(b1)

            process(j, t)
        return carry

    lax.fori_loop(0, NCH // NBUF, chunk3, 0)
    for b in range(NBUF):
        wait_scatter(b)
    plsc.subcore_barrier()

    def wcp(k, carry):
        q = s + k * NS
        pltpu.sync_copy(acc.at[pl.ds(q * WR, WR)], agg.at[c, pl.ds(q * WR, WR)])
        return carry

    lax.fori_loop(0, (NWCH - s + NS - 1) // NS, wcp, 0)


BPW = B // (NC * NS)


@functools.partial(
    pl.kernel,
    out_type=jax.ShapeDtypeStruct((B, D), jnp.float32),
    mesh=_mesh,
    scratch_types=[
        pltpu.VMEM((BPW,), jnp.int32),
        pltpu.VMEM((BPW, D), jnp.float32),
        pltpu.SemaphoreType.DMA,
    ],
)
def _batch_gather(table, idx, out, idx_v, rows_v, sem):
    wid = lax.axis_index("s") * NC + lax.axis_index("c")
    base = wid * BPW
    pltpu.sync_copy(idx.at[pl.ds(base, BPW)], idx_v)
    pltpu.async_copy(table.at[idx_v], rows_v, sem).wait()
    pltpu.sync_copy(rows_v, out.at[pl.ds(base, BPW)])


TN = 1000


def _mm1_body(a_ref, w_ref, b_ref, o_ref):
    a = a_ref[...]
    x = jnp.concatenate([a[0], a[1]], axis=1)
    y = jnp.dot(x, w_ref[...], preferred_element_type=jnp.float32) + b_ref[...]
    y = jnp.maximum(y, 0.0).astype(jnp.bfloat16)
    o_ref[0] = y[:, :H]
    o_ref[1] = y[:, H:]


_mm1 = pl.pallas_call(
    _mm1_body,
    grid=(N_NODES // TN,),
    in_specs=[
        pl.BlockSpec((NC, TN, H), lambda i: (0, i, 0)),
        pl.BlockSpec((D, D), lambda i: (0, 0)),
        pl.BlockSpec((1, D), lambda i: (0, 0)),
    ],
    out_specs=pl.BlockSpec((NC, TN, H), lambda i: (0, i, 0)),
    out_shape=jax.ShapeDtypeStruct((NC, N_NODES, H), jnp.bfloat16),
)


def _mm2_body(a_ref, pa_ref, w_ref, pb_ref, b_ref, o_ref):
    a = a_ref[...]
    x = jnp.concatenate([a[0], a[1]], axis=1)
    y = jnp.dot(x, w_ref[...], preferred_element_type=jnp.float32)
    y = y + jnp.dot(pa_ref[...], pb_ref[...], preferred_element_type=jnp.float32)
    o_ref[...] = y + b_ref[...]


_mm2 = pl.pallas_call(
    _mm2_body,
    grid=(N_NODES // TN,),
    in_specs=[
        pl.BlockSpec((NC, TN, H), lambda i: (0, i, 0)),
        pl.BlockSpec((TN, RANK), lambda i: (i, 0)),
        pl.BlockSpec((D, D), lambda i: (0, 0)),
        pl.BlockSpec((RANK, D), lambda i: (0, 0)),
        pl.BlockSpec((1, D), lambda i: (0, 0)),
    ],
    out_specs=pl.BlockSpec((TN, D), lambda i: (i, 0)),
    out_shape=jax.ShapeDtypeStruct((N_NODES, D), jnp.float32),
)


SB = 1024


def _sel_body(iv_ref, g_ref, base_ref, oov_ref, o_ref):
    m = iv_ref[...] > 0
    o_ref[...] = jnp.where(m, g_ref[...], base_ref[...] + oov_ref[...])


_sel = pl.pallas_call(
    _sel_body,
    grid=(B // SB,),
    in_specs=[
        pl.BlockSpec((SB, 1), lambda i: (i, 0)),
        pl.BlockSpec((SB, D), lambda i: (i, 0)),
        pl.BlockSpec((SB, D), lambda i: (i, 0)),
        pl.BlockSpec((1, D), lambda i: (0, 0)),
    ],
    out_specs=pl.BlockSpec((SB, D), lambda i: (i, 0)),
    out_shape=jax.ShapeDtypeStruct((B, D), jnp.float32),
)


_PERM128 = np.arange(128).reshape(4, 2, 16).transpose(0, 2, 1).reshape(128)
_P256 = np.concatenate([_PERM128, 128 + _PERM128])


def kernel(base_embedding, node_idx, in_vocab, edge_index, edge_weight, emb,
           W1, b1, W2, b2, pert_A, pert_B, oov_weight):
    src = edge_index[0].astype(jnp.int32)
    dst = edge_index[1].astype(jnp.int32)
    pad = EPTP - EPT
    packed = jnp.stack([src, dst], axis=0).reshape(2, NS, EPT)
    packed = jnp.pad(packed, ((0, 0), (0, 0), (0, pad)))
    packed = packed.reshape(2, NS, NCH, C)
    packed = jnp.transpose(packed, (1, 2, 0, 3))
    off = jnp.array([N_NODES, 0], jnp.int32)[None, None, :, None]
    edges = jnp.stack([packed, packed + off], axis=0)
    ws = jnp.pad(edge_weight.reshape(NS, EPT),
                 ((0, 0), (0, pad))).reshape(NS, NCH, C)

    emb2 = jnp.concatenate([emb[:, _P256[:H]], emb[:, _P256[H:]]],
                           axis=0).astype(jnp.bfloat16)
    W1p = W1[_P256, :][:, _P256]
    b1p = b1[_P256]
    W2p = W2[_P256, :]

    zrows = jnp.zeros((ZR, H), jnp.float32)

    agg1 = _edge_agg(emb2, edges, ws, zrows)
    h1 = _mm1(agg1, W1p, b1p.reshape(1, D))
    agg2 = _edge_agg(h1.reshape(NC * N_NODES, H), edges, ws, zrows)
    h2 = _mm2(agg2, pert_A, W2p, pert_B, b2.reshape(1, D))
    g = _batch_gather(h2, node_idx.astype(jnp.int32))
    out = _sel(in_vocab.astype(jnp.int32).reshape(B, 1), g,
               base_embedding, oov_weight)
    return out


# speedup vs baseline: 1.2335x; 1.0204x over previous
import functools

import numpy as np
import jax
import jax.numpy as jnp
from jax import lax
from jax.experimental import pallas as pl
from jax.experimental.pallas import tpu as pltpu
from jax.experimental.pallas import tpu_sc as plsc

N_NODES = 10000
N_EDGES = 160000
D = 256
H = 128
RANK = 32
B = 4096

NC = 2
NS = 16
L = 16

EPT = N_EDGES // NS
C = 128
NBUF = 3
NCH = 81
EPTP = NCH * C
WSCALE = float(2 ** -24)
ZR = 400
NZCH = N_NODES // ZR
WR = 80
NWCH = N_NODES // WR

_mesh = plsc.VectorSubcoreMesh(core_axis_name="c", subcore_axis_name="s")


@functools.partial(
    pl.kernel,
    out_type=jax.ShapeDtypeStruct((NC, N_NODES, H), jnp.bfloat16),
    mesh=_mesh,
    scratch_types=[
        pltpu.VMEM((NBUF, 3, C), jnp.int32),
        pltpu.VMEM((NBUF, C), jnp.int32),
        pltpu.VMEM((NBUF, C, H), jnp.bfloat16),
        pltpu.VMEM_SHARED((N_NODES, H), jnp.bfloat16),
        pltpu.SemaphoreType.DMA((NBUF,)),
        pltpu.SemaphoreType.DMA((NBUF,)),
        pltpu.SemaphoreType.DMA((NBUF,)),
    ],
    compiler_params=pltpu.CompilerParams(use_tc_tiling_on_sc=False),
)
def _edge_agg(x2, edges, zrows, agg, ec_v, dst_v, rows_v, acc,
              esem, gsem, ssem):
    c = lax.axis_index("c")
    s = lax.axis_index("s")

    def zcp(k, carry):
        q = s + k * NS
        pltpu.sync_copy(zrows, acc.at[pl.ds(q * ZR, ZR)])
        return carry

    lax.fori_loop(0, (NZCH - s + NS - 1) // NS, zcp, 0)
    plsc.subcore_barrier()

    def start_ec(j, b):
        pltpu.async_copy(edges.at[c, s, j], ec_v.at[b], esem.at[b])

    def wait_ec(b):
        pltpu.make_async_copy(edges.at[0, 0, 0], ec_v.at[b],
                              esem.at[b]).wait()

    def start_gather(b):
        pltpu.async_copy(x2.at[ec_v.at[b, 0]], rows_v.at[b], gsem.at[b])

    def wait_gather(b):
        pltpu.make_async_copy(x2.at[pl.ds(0, C)], rows_v.at[b],
                              gsem.at[b]).wait()

    def wait_scatter(b):
        pltpu.make_async_copy(rows_v.at[b], acc.at[pl.ds(0, C)],
                              ssem.at[b]).wait()

    start_ec(0, 0)
    start_ec(1, 1)
    wait_ec(0)
    start_gather(0)

    def process(j, b):
        wait_gather(b)

        for k in range(C // L):
            dst_v[b, pl.ds(k * L, L)] = ec_v[b, 1, pl.ds(k * L, L)]

        for e32 in range(C // (2 * L)):
            wi = ec_v[b, 2, pl.ds(e32 * 2 * L, 2 * L)]
            w32 = wi.astype(jnp.float32) * WSCALE
            for l in range(2 * L):
                wsp = w32.at[jnp.full((2 * L,), l, jnp.int32)].get(
                    mode="promise_in_bounds")
                wb = wsp.astype(jnp.bfloat16)
                e = e32 * 2 * L + l
                for k in range(H // (2 * L)):
                    sl = pl.ds(k * 2 * L, 2 * L)
                    rows_v[b, e, sl] = rows_v[b, e, sl] * wb

        pltpu.async_copy(rows_v.at[b], acc.at[dst_v.at[b]], ssem.at[b],
                         add=True)

    def chunk3(i, carry):
        for t in range(NBUF):
            j = NBUF * i + t
            b1 = (t + 1) % NBUF
            b2 = (t + 2) % NBUF

            @pl.when(j >= 2)
            def _(b1=b1):
                wait_scatter(b1)

            @pl.when(j + 1 < NCH)
            def _(b1=b1):
                wait_ec(b1)
                start_gather(b1)

            @pl.when(j + 2 < NCH)
            def _(j=j, b2=b2):
                start_ec(j + 2, b2)

            process(j, t)
        return carry

    lax.fori_loop(0, NCH // NBUF, chunk3, 0)
    wait_scatter((NCH - 2) % NBUF)
    wait_scatter((NCH - 1) % NBUF)
    plsc.subcore_barrier()

    def wcp(k, carry):
        q = s + k * NS
        pltpu.sync_copy(acc.at[pl.ds(q * WR, WR)], agg.at[c, pl.ds(q * WR, WR)])
        return carry

    lax.fori_loop(0, (NWCH - s + NS - 1) // NS, wcp, 0)


BPW = B // (NC * NS)


@functools.partial(
    pl.kernel,
    out_type=jax.ShapeDtypeStruct((B, D), jnp.float32),
    mesh=_mesh,
    scratch_types=[
        pltpu.VMEM((BPW,), jnp.int32),
        pltpu.VMEM((BPW, D), jnp.float32),
        pltpu.SemaphoreType.DMA,
    ],
)
def _batch_gather(table, idx, out, idx_v, rows_v, sem):
    wid = lax.axis_index("s") * NC + lax.axis_index("c")
    base = wid * BPW
    pltpu.sync_copy(idx.at[pl.ds(base, BPW)], idx_v)
    pltpu.async_copy(table.at[idx_v], rows_v, sem).wait()
    pltpu.sync_copy(rows_v, out.at[pl.ds(base, BPW)])


TN = 1000


def _mm1_body(a_ref, w_ref, b_ref, o_ref):
    a = a_ref[...]
    x = jnp.concatenate([a[0], a[1]], axis=1).astype(jnp.float32)
    y = jnp.dot(x, w_ref[...], preferred_element_type=jnp.float32) + b_ref[...]
    y = jnp.maximum(y, 0.0).astype(jnp.bfloat16)
    o_ref[0] = y[:, :H]
    o_ref[1] = y[:, H:]


_mm1 = pl.pallas_call(
    _mm1_body,
    grid=(N_NODES // TN,),
    in_specs=[
        pl.BlockSpec((NC, TN, H), lambda i: (0, i, 0)),
        pl.BlockSpec((D, D), lambda i: (0, 0)),
        pl.BlockSpec((1, D), lambda i: (0, 0)),
    ],
    out_specs=pl.BlockSpec((NC, TN, H), lambda i: (0, i, 0)),
    out_shape=jax.ShapeDtypeStruct((NC, N_NODES, H), jnp.bfloat16),
)


def _mm2_body(a_ref, pa_ref, w_ref, pb_ref, b_ref, o_ref):
    a = a_ref[...]
    x = jnp.concatenate([a[0], a[1]], axis=1).astype(jnp.float32)
    y = jnp.dot(x, w_ref[...], preferred_element_type=jnp.float32)
    y = y + jnp.dot(pa_ref[...], pb_ref[...], preferred_element_type=jnp.float32)
    o_ref[...] = y + b_ref[...]


_mm2 = pl.pallas_call(
    _mm2_body,
    grid=(N_NODES // TN,),
    in_specs=[
        pl.BlockSpec((NC, TN, H), lambda i: (0, i, 0)),
        pl.BlockSpec((TN, RANK), lambda i: (i, 0)),
        pl.BlockSpec((D, D), lambda i: (0, 0)),
        pl.BlockSpec((RANK, D), lambda i: (0, 0)),
        pl.BlockSpec((1, D), lambda i: (0, 0)),
    ],
    out_specs=pl.BlockSpec((TN, D), lambda i: (i, 0)),
    out_shape=jax.ShapeDtypeStruct((N_NODES, D), jnp.float32),
)


SB = 1024


def _sel_body(iv_ref, g_ref, base_ref, oov_ref, o_ref):
    m = iv_ref[...] > 0
    o_ref[...] = jnp.where(m, g_ref[...], base_ref[...] + oov_ref[...])


_sel = pl.pallas_call(
    _sel_body,
    grid=(B // SB,),
    in_specs=[
        pl.BlockSpec((SB, 1), lambda i: (i, 0)),
        pl.BlockSpec((SB, D), lambda i: (i, 0)),
        pl.BlockSpec((SB, D), lambda i: (i, 0)),
        pl.BlockSpec((1, D), lambda i: (0, 0)),
    ],
    out_specs=pl.BlockSpec((SB, D), lambda i: (i, 0)),
    out_shape=jax.ShapeDtypeStruct((B, D), jnp.float32),
)


def kernel(base_embedding, node_idx, in_vocab, edge_index, edge_weight, emb,
           W1, b1, W2, b2, pert_A, pert_B, oov_weight):
    src = edge_index[0].astype(jnp.int32)
    dst = edge_index[1].astype(jnp.int32)
    wm = jnp.round(edge_weight * (2.0 ** 24)).astype(jnp.int32)
    pad = EPTP - EPT
    packed = jnp.stack([src, dst, wm], axis=0).reshape(3, NS, EPT)
    packed = jnp.pad(packed, ((0, 0), (0, 0), (0, pad)))
    packed = packed.reshape(3, NS, NCH, C)
    packed = jnp.transpose(packed, (1, 2, 0, 3))
    off = jnp.array([N_NODES, 0, 0], jnp.int32)[None, None, :, None]
    edges = jnp.stack([packed, packed + off], axis=0)

    emb2 = jnp.concatenate([emb[:, :H], emb[:, H:]],
                           axis=0).astype(jnp.bfloat16)

    zrows = jnp.zeros((ZR, H), jnp.bfloat16)

    agg1 = _edge_agg(emb2, edges, zrows)
    h1 = _mm1(agg1, W1, b1.reshape(1, D))
    agg2 = _edge_agg(h1.reshape(NC * N_NODES, H), edges, zrows)
    h2 = _mm2(agg2, pert_A, W2, pert_B, b2.reshape(1, D))
    g = _batch_gather(h2, node_idx.astype(jnp.int32))
    out = _sel(in_vocab.astype(jnp.int32).reshape(B, 1), g,
               base_embedding, oov_weight)
    return out


# speedup vs baseline: 1.4293x; 1.1587x over previous
import functools

import numpy as np
import jax
import jax.numpy as jnp
from jax import lax
from jax.experimental import pallas as pl
from jax.experimental.pallas import tpu as pltpu
from jax.experimental.pallas import tpu_sc as plsc

N_NODES = 10000
N_EDGES = 160000
D = 256
H = 128
RANK = 32
B = 4096

NC = 2
NS = 16
L = 16

EPT = N_EDGES // NS
C = 96
NBUF = 3
NCH = 105
EPTP = NCH * C
WSCALE = float(2 ** -24)
ZR = 400
NZCH = N_NODES // ZR
WR = 80
NWCH = N_NODES // WR

_mesh = plsc.VectorSubcoreMesh(core_axis_name="c", subcore_axis_name="s")


@functools.partial(
    pl.kernel,
    out_type=jax.ShapeDtypeStruct((NC, N_NODES, H), jnp.float32),
    mesh=_mesh,
    scratch_types=[
        pltpu.VMEM((NBUF, 3, C), jnp.int32),
        pltpu.VMEM((NBUF, C), jnp.int32),
        pltpu.VMEM((NBUF, C, H), jnp.float32),
        pltpu.VMEM_SHARED((N_NODES, H), jnp.float32),
        pltpu.SemaphoreType.DMA((NBUF,)),
        pltpu.SemaphoreType.DMA((NBUF,)),
        pltpu.SemaphoreType.DMA((NBUF,)),
    ],
)
def _edge_agg(x2, edges, zrows, agg, ec_v, dst_v, rows_v, acc,
              esem, gsem, ssem):
    c = lax.axis_index("c")
    s = lax.axis_index("s")

    def zcp(k, carry):
        q = s + k * NS
        pltpu.sync_copy(zrows, acc.at[pl.ds(q * ZR, ZR)])
        return carry

    lax.fori_loop(0, (NZCH - s + NS - 1) // NS, zcp, 0)
    plsc.subcore_barrier()

    def start_ec(j, b):
        pltpu.async_copy(edges.at[c, s, j], ec_v.at[b], esem.at[b])

    def wait_ec(b):
        pltpu.make_async_copy(edges.at[0, 0, 0], ec_v.at[b],
                              esem.at[b]).wait()

    def start_gather(b):
        pltpu.async_copy(x2.at[ec_v.at[b, 0]], rows_v.at[b], gsem.at[b])

    def wait_gather(b):
        pltpu.make_async_copy(x2.at[pl.ds(0, C)], rows_v.at[b],
                              gsem.at[b]).wait()

    def wait_scatter(b):
        pltpu.make_async_copy(rows_v.at[b], acc.at[pl.ds(0, C)],
                              ssem.at[b]).wait()

    start_ec(0, 0)
    start_ec(1, 1)
    wait_ec(0)
    start_gather(0)

    def process(j, b):
        wait_gather(b)

        for k in range(C // L):
            dst_v[b, pl.ds(k * L, L)] = ec_v[b, 1, pl.ds(k * L, L)]

        for e32 in range(C // (2 * L)):
            wi = ec_v[b, 2, pl.ds(e32 * 2 * L, 2 * L)]
            w32 = wi.astype(jnp.float32) * WSCALE
            for l in range(2 * L):
                wsp = w32.at[jnp.full((2 * L,), l, jnp.int32)].get(
                    mode="promise_in_bounds")
                e = e32 * 2 * L + l
                for k in range(H // (2 * L)):
                    sl = pl.ds(k * 2 * L, 2 * L)
                    rows_v[b, e, sl] = rows_v[b, e, sl] * wsp

        pltpu.async_copy(rows_v.at[b], acc.at[dst_v.at[b]], ssem.at[b],
                         add=True)

    def chunk3(i, carry):
        for t in range(NBUF):
            j = NBUF * i + t
            b1 = (t + 1) % NBUF
            b2 = (t + 2) % NBUF

            @pl.when(j >= 2)
            def _(b1=b1):
                wait_scatter(b1)

            @pl.when(j + 1 < NCH)
            def _(b1=b1):
                wait_ec(b1)
                start_gather(b1)

            @pl.when(j + 2 < NCH)
            def _(j=j, b2=b2):
                start_ec(j + 2, b2)

            process(j, t)
        return carry

    lax.fori_loop(0, NCH // NBUF, chunk3, 0)
    wait_scatter((NCH - 2) % NBUF)
    wait_scatter((NCH - 1) % NBUF)
    plsc.subcore_barrier()

    def wcp(k, carry):
        q = s + k * NS
        pltpu.sync_copy(acc.at[pl.ds(q * WR, WR)], agg.at[c, pl.ds(q * WR, WR)])
        return carry

    lax.fori_loop(0, (NWCH - s + NS - 1) // NS, wcp, 0)


BPW = B // (NC * NS)


@functools.partial(
    pl.kernel,
    out_type=jax.ShapeDtypeStruct((B, D), jnp.float32),
    mesh=_mesh,
    scratch_types=[
        pltpu.VMEM((BPW,), jnp.int32),
        pltpu.VMEM((BPW, D), jnp.float32),
        pltpu.SemaphoreType.DMA,
    ],
)
def _batch_gather(table, idx, out, idx_v, rows_v, sem):
    wid = lax.axis_index("s") * NC + lax.axis_index("c")
    base = wid * BPW
    pltpu.sync_copy(idx.at[pl.ds(base, BPW)], idx_v)
    pltpu.async_copy(table.at[idx_v], rows_v, sem).wait()
    pltpu.sync_copy(rows_v, out.at[pl.ds(base, BPW)])


TN = 1000


def _mm1_body(a_ref, w_ref, b_ref, o_ref):
    a = a_ref[...]
    x = jnp.concatenate([a[0], a[1]], axis=1).astype(jnp.float32)
    y = jnp.dot(x, w_ref[...], preferred_element_type=jnp.float32) + b_ref[...]
    y = jnp.maximum(y, 0.0)
    o_ref[0] = y[:, :H]
    o_ref[1] = y[:, H:]


_mm1 = pl.pallas_call(
    _mm1_body,
    grid=(N_NODES // TN,),
    in_specs=[
        pl.BlockSpec((NC, TN, H), lambda i: (0, i, 0)),
        pl.BlockSpec((D, D), lambda i: (0, 0)),
        pl.BlockSpec((1, D), lambda i: (0, 0)),
    ],
    out_specs=pl.BlockSpec((NC, TN, H), lambda i: (0, i, 0)),
    out_shape=jax.ShapeDtypeStruct((NC, N_NODES, H), jnp.float32),
)


def _mm2_body(a_ref, pa_ref, w_ref, pb_ref, b_ref, o_ref):
    a = a_ref[...]
    x = jnp.concatenate([a[0], a[1]], axis=1).astype(jnp.float32)
    y = jnp.dot(x, w_ref[...], preferred_element_type=jnp.float32)
    y = y + jnp.dot(pa_ref[...], pb_ref[...], preferred_element_type=jnp.float32)
    o_ref[...] = y + b_ref[...]


_mm2 = pl.pallas_call(
    _mm2_body,
    grid=(N_NODES // TN,),
    in_specs=[
        pl.BlockSpec((NC, TN, H), lambda i: (0, i, 0)),
        pl.BlockSpec((TN, RANK), lambda i: (i, 0)),
        pl.BlockSpec((D, D), lambda i: (0, 0)),
        pl.BlockSpec((RANK, D), lambda i: (0, 0)),
        pl.BlockSpec((1, D), lambda i: (0, 0)),
    ],
    out_specs=pl.BlockSpec((TN, D), lambda i: (i, 0)),
    out_shape=jax.ShapeDtypeStruct((N_NODES, D), jnp.float32),
)


SB = 1024


def _sel_body(iv_ref, g_ref, base_ref, oov_ref, o_ref):
    m = iv_ref[...] > 0
    o_ref[...] = jnp.where(m, g_ref[...], base_ref[...] + oov_ref[...])


_sel = pl.pallas_call(
    _sel_body,
    grid=(B // SB,),
    in_specs=[
        pl.BlockSpec((SB, 1), lambda i: (i, 0)),
        pl.BlockSpec((SB, D), lambda i: (i, 0)),
        pl.BlockSpec((SB, D), lambda i: (i, 0)),
        pl.BlockSpec((1, D), lambda i: (0, 0)),
    ],
    out_specs=pl.BlockSpec((SB, D), lambda i: (i, 0)),
    out_shape=jax.ShapeDtypeStruct((B, D), jnp.float32),
)


def kernel(base_embedding, node_idx, in_vocab, edge_index, edge_weight, emb,
           W1, b1, W2, b2, pert_A, pert_B, oov_weight):
    src = edge_index[0].astype(jnp.int32)
    dst = edge_index[1].astype(jnp.int32)
    wm = jnp.round(edge_weight * (2.0 ** 24)).astype(jnp.int32)
    pad = EPTP - EPT
    packed = jnp.stack([src, dst, wm], axis=0).reshape(3, NS, EPT)
    packed = jnp.pad(packed, ((0, 0), (0, 0), (0, pad)))
    packed = packed.reshape(3, NS, NCH, C)
    packed = jnp.transpose(packed, (1, 2, 0, 3))
    off = jnp.array([N_NODES, 0, 0], jnp.int32)[None, None, :, None]
    edges = jnp.stack([packed, packed + off], axis=0)

    emb2 = jnp.concatenate([emb[:, :H], emb[:, H:]], axis=0)

    zrows = jnp.zeros((ZR, H), jnp.float32)

    agg1 = _edge_agg(emb2, edges, zrows)
    h1 = _mm1(agg1, W1, b1.reshape(1, D))
    agg2 = _edge_agg(h1.reshape(NC * N_NODES, H), edges, zrows)
    h2 = _mm2(agg2, pert_A, W2, pert_B, b2.reshape(1, D))
    g = _batch_gather(h2, node_idx.astype(jnp.int32))
    out = _sel(in_vocab.astype(jnp.int32).reshape(B, 1), g,
               base_embedding, oov_weight)
    return out


# speedup vs baseline: 1.4339x; 1.0032x over previous
import functools

import jax
import jax.numpy as jnp
from jax import lax
from jax.experimental import pallas as pl
from jax.experimental.pallas import tpu as pltpu
from jax.experimental.pallas import tpu_sc as plsc

N_NODES = 10000
N_EDGES = 160000
D = 256
H = 128
RANK = 32
B = 4096

NC = 2
NS = 16
L = 16

EPT = N_EDGES // NS
C = 96
NBUF = 3
NCH = 105
EPTP = NCH * C
WSCALE = float(2 ** -24)
ZR = 400
NZCH = N_NODES // ZR
WR = 80
NWCH = N_NODES // WR

_mesh = plsc.VectorSubcoreMesh(core_axis_name="c", subcore_axis_name="s")


@functools.partial(
    pl.kernel,
    out_type=jax.ShapeDtypeStruct((NC, N_NODES, H), jnp.float32),
    mesh=_mesh,
    scratch_types=[
        pltpu.VMEM((NBUF, 3, C), jnp.int32),
        pltpu.VMEM((NBUF, C), jnp.int32),
        pltpu.VMEM((NBUF, C, H), jnp.float32),
        pltpu.VMEM_SHARED((N_NODES, H), jnp.float32),
        pltpu.SemaphoreType.DMA((NBUF,)),
        pltpu.SemaphoreType.DMA((NBUF,)),
        pltpu.SemaphoreType.DMA((NBUF,)),
    ],
)
def _edge_agg(x2, edges, zrows, agg, ec_v, dst_v, rows_v, acc,
              esem, gsem, ssem):
    c = lax.axis_index("c")
    s = lax.axis_index("s")

    def zcp(k, carry):
        q = s + k * NS
        pltpu.sync_copy(zrows, acc.at[pl.ds(q * ZR, ZR)])
        return carry

    lax.fori_loop(0, (NZCH - s + NS - 1) // NS, zcp, 0)
    plsc.subcore_barrier()

    def start_ec(j, b):
        pltpu.async_copy(edges.at[c, s, j], ec_v.at[b], esem.at[b])

    def wait_ec(b):
        pltpu.make_async_copy(edges.at[0, 0, 0], ec_v.at[b],
                              esem.at[b]).wait()

    def start_gather(b):
        pltpu.async_copy(x2.at[ec_v.at[b, 0]], rows_v.at[b], gsem.at[b])

    def wait_gather(b):
        pltpu.make_async_copy(x2.at[pl.ds(0, C)], rows_v.at[b],
                              gsem.at[b]).wait()

    def wait_scatter(b):
        pltpu.make_async_copy(rows_v.at[b], acc.at[pl.ds(0, C)],
                              ssem.at[b]).wait()

    start_ec(0, 0)
    start_ec(1, 1)
    wait_ec(0)
    start_gather(0)

    def process(j, b):
        wait_gather(b)

        for k in range(C // L):
            dst_v[b, pl.ds(k * L, L)] = ec_v[b, 1, pl.ds(k * L, L)]

        for e32 in range(C // (2 * L)):
            wi = ec_v[b, 2, pl.ds(e32 * 2 * L, 2 * L)]
            w32 = wi.astype(jnp.float32) * WSCALE
            for l in range(2 * L):
                wsp = w32.at[jnp.full((2 * L,), l, jnp.int32)].get(
                    mode="promise_in_bounds")
                e = e32 * 2 * L + l
                for k in range(H // (2 * L)):
                    sl = pl.ds(k * 2 * L, 2 * L)
                    rows_v[b, e, sl] = rows_v[b, e, sl] * wsp

        pltpu.async_copy(rows_v.at[b], acc.at[dst_v.at[b]], ssem.at[b],
                         add=True)

    def chunk3(i, carry):
        for t in range(NBUF):
            j = NBUF * i + t
            b1 = (t + 1) % NBUF
            b2 = (t + 2) % NBUF

            @pl.when(j >= 2)
            def _(b1=b1):
                wait_scatter(b1)

            @pl.when(j + 1 < NCH)
            def _(b1=b1):
                wait_ec(b1)
                start_gather(b1)

            @pl.when(j + 2 < NCH)
            def _(j=j, b2=b2):
                start_ec(j + 2, b2)

            process(j, t)
        return carry

    lax.fori_loop(0, NCH // NBUF, chunk3, 0)
    wait_scatter((NCH - 2) % NBUF)
    wait_scatter((NCH - 1) % NBUF)
    plsc.subcore_barrier()

    def wcp(k, carry):
        q = s + k * NS
        pltpu.sync_copy(acc.at[pl.ds(q * WR, WR)], agg.at[c, pl.ds(q * WR, WR)])
        return carry

    lax.fori_loop(0, (NWCH - s + NS - 1) // NS, wcp, 0)


BPW = B // (NC * NS)


@functools.partial(
    pl.kernel,
    out_type=jax.ShapeDtypeStruct((B, D), jnp.float32),
    mesh=_mesh,
    scratch_types=[
        pltpu.VMEM((BPW,), jnp.int32),
        pltpu.VMEM((BPW, D), jnp.float32),
        pltpu.SemaphoreType.DMA,
    ],
)
def _batch_gather(table, idx, out, idx_v, rows_v, sem):
    wid = lax.axis_index("s") * NC + lax.axis_index("c")
    base = wid * BPW
    pltpu.sync_copy(idx.at[pl.ds(base, BPW)], idx_v)
    pltpu.async_copy(table.at[idx_v], rows_v, sem).wait()
    pltpu.sync_copy(rows_v, out.at[pl.ds(base, BPW)])


TN = 1000


def _mm1_body(a_ref, w_ref, b_ref, o_ref):
    a = a_ref[...]
    x = jnp.concatenate([a[0], a[1]], axis=1).astype(jnp.float32)
    y = jnp.dot(x, w_ref[...], preferred_element_type=jnp.float32) + b_ref[...]
    y = jnp.maximum(y, 0.0)
    o_ref[0] = y[:, :H]
    o_ref[1] = y[:, H:]


_mm1 = pl.pallas_call(
    _mm1_body,
    grid=(N_NODES // TN,),
    in_specs=[
        pl.BlockSpec((NC, TN, H), lambda i: (0, i, 0)),
        pl.BlockSpec((D, D), lambda i: (0, 0)),
        pl.BlockSpec((1, D), lambda i: (0, 0)),
    ],
    out_specs=pl.BlockSpec((NC, TN, H), lambda i: (0, i, 0)),
    out_shape=jax.ShapeDtypeStruct((NC, N_NODES, H), jnp.float32),
)


def _mm2_body(a_ref, pa_ref, w_ref, pb_ref, b_ref, o_ref):
    a = a_ref[...]
    x = jnp.concatenate([a[0], a[1]], axis=1).astype(jnp.float32)
    y = jnp.dot(x, w_ref[...], preferred_element_type=jnp.float32)
    y = y + jnp.dot(pa_ref[...], pb_ref[...], preferred_element_type=jnp.float32)
    o_ref[...] = y + b_ref[...]


_mm2 = pl.pallas_call(
    _mm2_body,
    grid=(N_NODES // TN,),
    in_specs=[
        pl.BlockSpec((NC, TN, H), lambda i: (0, i, 0)),
        pl.BlockSpec((TN, RANK), lambda i: (i, 0)),
        pl.BlockSpec((D, D), lambda i: (0, 0)),
        pl.BlockSpec((RANK, D), lambda i: (0, 0)),
        pl.BlockSpec((1, D), lambda i: (0, 0)),
    ],
    out_specs=pl.BlockSpec((TN, D), lambda i: (i, 0)),
    out_shape=jax.ShapeDtypeStruct((N_NODES, D), jnp.float32),
)


SB = 1024


def _sel_body(iv_ref, g_ref, base_ref, oov_ref, o_ref):
    m = iv_ref[...] > 0
    o_ref[...] = jnp.where(m, g_ref[...], base_ref[...] + oov_ref[...])


_sel = pl.pallas_call(
    _sel_body,
    grid=(B // SB,),
    in_specs=[
        pl.BlockSpec((SB, 1), lambda i: (i, 0)),
        pl.BlockSpec((SB, D), lambda i: (i, 0)),
        pl.BlockSpec((SB, D), lambda i: (i, 0)),
        pl.BlockSpec((1, D), lambda i: (0, 0)),
    ],
    out_specs=pl.BlockSpec((SB, D), lambda i: (i, 0)),
    out_shape=jax.ShapeDtypeStruct((B, D), jnp.float32),
)


def kernel(base_embedding, node_idx, in_vocab, edge_index, edge_weight, emb,
           W1, b1, W2, b2, pert_A, pert_B, oov_weight):
    src = edge_index[0].astype(jnp.int32)
    dst = edge_index[1].astype(jnp.int32)
    wm = jnp.round(edge_weight * (2.0 ** 24)).astype(jnp.int32)
    pad = EPTP - EPT
    packed = jnp.stack([src, dst, wm], axis=0).reshape(3, NS, EPT)
    packed = jnp.pad(packed, ((0, 0), (0, 0), (0, pad)))
    packed = packed.reshape(3, NS, NCH, C)
    packed = jnp.transpose(packed, (1, 2, 0, 3))
    off = jnp.array([N_NODES, 0, 0], jnp.int32)[None, None, :, None]
    edges = jnp.stack([packed, packed + off], axis=0)

    emb2 = jnp.concatenate([emb[:, :H], emb[:, H:]], axis=0)

    zrows = jnp.zeros((ZR, H), jnp.float32)

    agg1 = _edge_agg(emb2, edges, zrows)
    h1 = _mm1(agg1, W1, b1.reshape(1, D))
    agg2 = _edge_agg(h1.reshape(NC * N_NODES, H), edges, zrows)
    h2 = _mm2(agg2, pert_A, W2, pert_B, b2.reshape(1, D))
    g = _batch_gather(h2, node_idx.astype(jnp.int32))
    out = _sel(in_vocab.astype(jnp.int32).reshape(B, 1), g,
               base_embedding, oov_weight)
    return out
